# Initial kernel scaffold; baseline (speedup 1.0000x reference)
import functools

import jax
import jax.numpy as jnp
from jax import lax
from jax.experimental import pallas as pl
from jax.experimental.pallas import tpu as pltpu
from jax.experimental.pallas import tpu_sc as plsc

N_NODES = 10000
N_EDGES = 320000
D = 128

NC = 2
NS = 16
NW = NC * NS
EPW = N_EDGES // NW
K = 80
NCHUNK = EPW // K
ROWS_PER_TILE = N_NODES // NS
ZROWS = 125
LANES = 16


def _splat(vec16, i):
    idx = jnp.full((LANES,), i, dtype=jnp.int32)
    return jnp.take(vec16, idx, mode="promise_in_bounds")


def _sc_body(feat_hbm, src_hbm, dst_hbm, w_hbm, out_hbm,
             src_v, dst_v, w_v, rows_v, zbuf, acc, sem):
    cid = lax.axis_index("c")
    sid = lax.axis_index("s")
    wid = sid * NC + cid

    zero16 = jnp.zeros((LANES,), jnp.float32)

    def zrow(r, carry):
        for j in range(D // LANES):
            zbuf[r, pl.ds(j * LANES, LANES)] = zero16
        return carry

    lax.fori_loop(0, ZROWS, zrow, 0)
    for z in range(ROWS_PER_TILE // ZROWS):
        pltpu.sync_copy(zbuf, acc.at[pl.ds(sid * ROWS_PER_TILE + z * ZROWS, ZROWS)])
    plsc.subcore_barrier()

    pltpu.sync_copy(src_hbm.at[wid], src_v)
    pltpu.sync_copy(dst_hbm.at[wid], dst_v)
    pltpu.sync_copy(w_hbm.at[wid], w_v)

    def chunk(c, carry):
        pltpu.async_copy(feat_hbm.at[src_v.at[c]], rows_v, sem).wait()
        for g in range(K // LANES):
            w16 = w_v[c, pl.ds(g * LANES, LANES)]
            for i in range(LANES):
                wi = _splat(w16, i)
                r = g * LANES + i
                for j in range(D // LANES):
                    rows_v[r, pl.ds(j * LANES, LANES)] = (
                        rows_v[r, pl.ds(j * LANES, LANES)] * wi)
        pltpu.sync_copy(rows_v, acc.at[dst_v.at[c]], add=True)
        return carry

    lax.fori_loop(0, NCHUNK, chunk, 0)
    plsc.subcore_barrier()

    pltpu.sync_copy(acc.at[pl.ds(sid * ROWS_PER_TILE, ROWS_PER_TILE)],
                    out_hbm.at[pl.ds(cid * N_NODES + sid * ROWS_PER_TILE,
                                     ROWS_PER_TILE)])


_sc_scatter = functools.partial(
    pl.kernel,
    mesh=plsc.VectorSubcoreMesh(core_axis_name="c", subcore_axis_name="s"),
    out_type=jax.ShapeDtypeStruct((NC * N_NODES, D), jnp.float32),
    scratch_types=[
        pltpu.VMEM((NCHUNK, K), jnp.int32),
        pltpu.VMEM((NCHUNK, K), jnp.int32),
        pltpu.VMEM((NCHUNK, K), jnp.float32),
        pltpu.VMEM((K, D), jnp.float32),
        pltpu.VMEM((ZROWS, D), jnp.float32),
        pltpu.VMEM_SHARED((N_NODES, D), jnp.float32),
        pltpu.SemaphoreType.DMA,
    ],
)(_sc_body)


def _tc_body(p_ref, w_ref, o_ref):
    x = p_ref[0] + p_ref[1]
    o_ref[...] = jnp.maximum(
        jnp.dot(x, w_ref[...], preferred_element_type=jnp.float32), 0.0)


M_BLK = 1000

_tc_matmul = pl.pallas_call(
    _tc_body,
    grid=(N_NODES // M_BLK,),
    in_specs=[pl.BlockSpec((NC, M_BLK, D), lambda m: (0, m, 0)),
              pl.BlockSpec((D, D), lambda m: (0, 0))],
    out_specs=pl.BlockSpec((M_BLK, D), lambda m: (m, 0)),
    out_shape=jax.ShapeDtypeStruct((N_NODES, D), jnp.float32),
)


def kernel(feat, edge_index, edge_weight, W):
    src = edge_index[0].astype(jnp.int32).reshape(NW, NCHUNK, K)
    dst = edge_index[1].astype(jnp.int32).reshape(NW, NCHUNK, K)
    w3 = edge_weight.reshape(NW, NCHUNK, K)
    partials = _sc_scatter(feat, src, dst, w3)
    return _tc_matmul(partials.reshape(NC, N_NODES, D), W)


# speedup vs baseline: 6.1356x; 6.1356x over previous
import functools

import jax
import jax.numpy as jnp
from jax import lax
from jax.experimental import pallas as pl
from jax.experimental.pallas import tpu as pltpu
from jax.experimental.pallas import tpu_sc as plsc

N_NODES = 10000
N_EDGES = 320000
D = 128

NC = 2
NS = 16
NW = NC * NS
EPW = N_EDGES // NW
K = 80
NCHUNK = EPW // K
NB = 25
NBLK = NCHUNK // NB
ROWS_PER_TILE = 624
TAIL_ROWS = N_NODES - NS * ROWS_PER_TILE
LANES = 16

_GATHER_DNUMS = lax.GatherDimensionNumbers(
    offset_dims=(), collapsed_slice_dims=(0,), start_index_map=(0,))


def _splat(vec16, i):
    idx = jnp.full((LANES, 1), i, dtype=jnp.int32)
    return lax.gather(vec16, idx, _GATHER_DNUMS, slice_sizes=(1,),
                      mode=lax.GatherScatterMode.PROMISE_IN_BOUNDS)


def _sc_body(feat_hbm, src_hbm, dst_hbm, w_hbm, out_hbm,
             src_v, dst_v, w_v, rows_v, acc, sem):
    cid = lax.axis_index("c")
    sid = lax.axis_index("s")
    wid = sid * NC + cid

    zero16 = jnp.zeros((LANES,), jnp.float32)

    def zrow(r, carry):
        for j in range(D // LANES):
            rows_v[r, pl.ds(j * LANES, LANES)] = zero16
        return carry

    lax.fori_loop(0, K, zrow, 0)
    zbase = sid * ROWS_PER_TILE
    for z in range(ROWS_PER_TILE // K):
        pltpu.sync_copy(rows_v, acc.at[pl.ds(zbase + z * K, K)])
    zrem = ROWS_PER_TILE - (ROWS_PER_TILE // K) * K
    pltpu.sync_copy(rows_v.at[pl.ds(0, zrem)],
                    acc.at[pl.ds(zbase + ROWS_PER_TILE - zrem, zrem)])

    @pl.when(sid == NS - 1)
    def _zero_tail():
        pltpu.sync_copy(rows_v.at[pl.ds(0, TAIL_ROWS)],
                        acc.at[pl.ds(NS * ROWS_PER_TILE, TAIL_ROWS)])

    plsc.subcore_barrier()

    def chunk(c, carry):
        pltpu.async_copy(feat_hbm.at[src_v.at[c]], rows_v, sem).wait()
        for g in range(K // LANES):
            w16 = w_v[c, pl.ds(g * LANES, LANES)]
            for i in range(LANES):
                wi = _splat(w16, i)
                r = g * LANES + i
                for j in range(D // LANES):
                    rows_v[r, pl.ds(j * LANES, LANES)] = (
                        rows_v[r, pl.ds(j * LANES, LANES)] * wi)
        pltpu.sync_copy(rows_v, acc.at[dst_v.at[c]], add=True)
        return carry

    def block(b, carry):
        pltpu.sync_copy(src_hbm.at[wid, b], src_v)
        pltpu.sync_copy(dst_hbm.at[wid, b], dst_v)
        pltpu.sync_copy(w_hbm.at[wid, b], w_v)
        lax.fori_loop(0, NB, chunk, 0)
        return carry

    lax.fori_loop(0, NBLK, block, 0)
    plsc.subcore_barrier()

    pltpu.sync_copy(acc.at[pl.ds(sid * ROWS_PER_TILE, ROWS_PER_TILE)],
                    out_hbm.at[pl.ds(cid * N_NODES + sid * ROWS_PER_TILE,
                                     ROWS_PER_TILE)])

    @pl.when(sid == NS - 1)
    def _write_tail():
        pltpu.sync_copy(acc.at[pl.ds(NS * ROWS_PER_TILE, TAIL_ROWS)],
                        out_hbm.at[pl.ds(cid * N_NODES + NS * ROWS_PER_TILE,
                                         TAIL_ROWS)])


_sc_scatter = functools.partial(
    pl.kernel,
    mesh=plsc.VectorSubcoreMesh(core_axis_name="c", subcore_axis_name="s"),
    out_type=jax.ShapeDtypeStruct((NC * N_NODES, D), jnp.float32),
    scratch_types=[
        pltpu.VMEM((NB, K), jnp.int32),
        pltpu.VMEM((NB, K), jnp.int32),
        pltpu.VMEM((NB, K), jnp.float32),
        pltpu.VMEM((K, D), jnp.float32),
        pltpu.VMEM_SHARED((N_NODES, D), jnp.float32),
        pltpu.SemaphoreType.DMA,
    ],
)(_sc_body)


def _tc_body(p_ref, w_ref, o_ref):
    x = p_ref[0] + p_ref[1]
    o_ref[...] = jnp.maximum(
        jnp.dot(x, w_ref[...], preferred_element_type=jnp.float32), 0.0)


M_BLK = 1000

_tc_matmul = pl.pallas_call(
    _tc_body,
    grid=(N_NODES // M_BLK,),
    in_specs=[pl.BlockSpec((NC, M_BLK, D), lambda m: (0, m, 0)),
              pl.BlockSpec((D, D), lambda m: (0, 0))],
    out_specs=pl.BlockSpec((M_BLK, D), lambda m: (m, 0)),
    out_shape=jax.ShapeDtypeStruct((N_NODES, D), jnp.float32),
)


def kernel(feat, edge_index, edge_weight, W):
    src = edge_index[0].astype(jnp.int32).reshape(NW, NBLK, NB, K)
    dst = edge_index[1].astype(jnp.int32).reshape(NW, NBLK, NB, K)
    w4 = edge_weight.reshape(NW, NBLK, NB, K)
    partials = _sc_scatter(feat, src, dst, w4)
    return _tc_matmul(partials.reshape(NC, N_NODES, D), W)


# speedup vs baseline: 9.8046x; 1.5980x over previous
import functools

import jax
import jax.numpy as jnp
from jax import lax
from jax.experimental import pallas as pl
from jax.experimental.pallas import tpu as pltpu
from jax.experimental.pallas import tpu_sc as plsc

N_NODES = 10000
N_EDGES = 320000
D = 128

NC = 2
NS = 16
NW = NC * NS
K = 64
NBCH = 27
NBLK = 6
EPW = NBLK * NBCH * K
E_PAD = NW * EPW
ROWS_PER_TILE = 624
TAIL_ROWS = N_NODES - NS * ROWS_PER_TILE
LANES = 16
NBUF = 3

_GATHER_DNUMS = lax.GatherDimensionNumbers(
    offset_dims=(), collapsed_slice_dims=(0,), start_index_map=(0,))


def _splat(vec16, i):
    idx = jnp.full((LANES, 1), i, dtype=jnp.int32)
    return lax.gather(vec16, idx, _GATHER_DNUMS, slice_sizes=(1,),
                      mode=lax.GatherScatterMode.PROMISE_IN_BOUNDS)


def _sc_body(feat_hbm, src_hbm, dst_hbm, w_hbm, out_hbm,
             src_v, dst_v, w_v, rows0, rows1, rows2, acc,
             gsem0, gsem1, gsem2, ssem0, ssem1, ssem2):
    cid = lax.axis_index("c")
    sid = lax.axis_index("s")
    wid = sid * NC + cid
    rows = (rows0, rows1, rows2)
    gsem = (gsem0, gsem1, gsem2)
    ssem = (ssem0, ssem1, ssem2)

    zero16 = jnp.zeros((LANES,), jnp.float32)

    def zrow(r, carry):
        for j in range(D // LANES):
            rows0[r, pl.ds(j * LANES, LANES)] = zero16
        return carry

    lax.fori_loop(0, K, zrow, 0)
    zbase = sid * ROWS_PER_TILE
    for z in range(ROWS_PER_TILE // K):
        pltpu.sync_copy(rows0, acc.at[pl.ds(zbase + z * K, K)])
    zrem = ROWS_PER_TILE - (ROWS_PER_TILE // K) * K
    pltpu.sync_copy(rows0.at[pl.ds(0, zrem)],
                    acc.at[pl.ds(zbase + ROWS_PER_TILE - zrem, zrem)])

    @pl.when(sid == NS - 1)
    def _zero_tail():
        pltpu.sync_copy(rows0.at[pl.ds(0, TAIL_ROWS)],
                        acc.at[pl.ds(NS * ROWS_PER_TILE, TAIL_ROWS)])

    plsc.subcore_barrier()

    def start_g(c, bi):
        pltpu.async_copy(feat_hbm.at[src_v.at[c]], rows[bi], gsem[bi])

    def wait_g(bi):
        pltpu.make_async_copy(feat_hbm.at[pl.ds(0, K)], rows[bi],
                              gsem[bi]).wait()

    def start_s(c, bi):
        pltpu.async_copy(rows[bi], acc.at[dst_v.at[c]], ssem[bi], add=True)

    def wait_s(bi):
        pltpu.make_async_copy(rows[bi], acc.at[pl.ds(0, K)], ssem[bi]).wait()

    def scale(bi, c):
        buf = rows[bi]

        def grp(g, carry):
            off = pl.multiple_of(g * LANES, LANES)
            w16 = w_v[c, pl.ds(off, LANES)]
            for i in range(LANES):
                wi = _splat(w16, i)
                r = g * LANES + i
                for j in range(D // LANES):
                    buf[r, pl.ds(j * LANES, LANES)] = (
                        buf[r, pl.ds(j * LANES, LANES)] * wi)
            return carry

        lax.fori_loop(0, K // LANES, grp, 0)

    def block(b, carry):
        pltpu.sync_copy(src_hbm.at[wid, b], src_v)
        pltpu.sync_copy(dst_hbm.at[wid, b], dst_v)
        pltpu.sync_copy(w_hbm.at[wid, b], w_v)

        start_g(0, 0)
        start_g(1, 1)
        wait_g(0)
        scale(0, 0)
        start_s(0, 0)
        start_g(2, 2)
        wait_g(1)
        scale(1, 1)
        start_s(1, 1)

        def tri(t, carry2):
            for j in range(3):
                c = 2 + 3 * t + j
                bi = (2 + j) % 3
                ni = (3 + j) % 3
                wait_s(ni)
                start_g(c + 1, ni)
                wait_g(bi)
                scale(bi, c)
                start_s(c, bi)
            return carry2

        lax.fori_loop(0, (NBCH - 3) // 3, tri, 0)

        wait_s(0)
        wait_g(2)
        scale(2, NBCH - 1)
        start_s(NBCH - 1, 2)
        wait_s(1)
        wait_s(2)
        return carry

    lax.fori_loop(0, NBLK, block, 0)
    plsc.subcore_barrier()

    pltpu.sync_copy(acc.at[pl.ds(sid * ROWS_PER_TILE, ROWS_PER_TILE)],
                    out_hbm.at[pl.ds(cid * N_NODES + sid * ROWS_PER_TILE,
                                     ROWS_PER_TILE)])

    @pl.when(sid == NS - 1)
    def _write_tail():
        pltpu.sync_copy(acc.at[pl.ds(NS * ROWS_PER_TILE, TAIL_ROWS)],
                        out_hbm.at[pl.ds(cid * N_NODES + NS * ROWS_PER_TILE,
                                         TAIL_ROWS)])


_sc_scatter = functools.partial(
    pl.kernel,
    mesh=plsc.VectorSubcoreMesh(core_axis_name="c", subcore_axis_name="s"),
    out_type=jax.ShapeDtypeStruct((NC * N_NODES, D), jnp.float32),
    scratch_types=[
        pltpu.VMEM((NBCH, K), jnp.int32),
        pltpu.VMEM((NBCH, K), jnp.int32),
        pltpu.VMEM((NBCH, K), jnp.float32),
        pltpu.VMEM((K, D), jnp.float32),
        pltpu.VMEM((K, D), jnp.float32),
        pltpu.VMEM((K, D), jnp.float32),
        pltpu.VMEM_SHARED((N_NODES, D), jnp.float32),
        pltpu.SemaphoreType.DMA,
        pltpu.SemaphoreType.DMA,
        pltpu.SemaphoreType.DMA,
        pltpu.SemaphoreType.DMA,
        pltpu.SemaphoreType.DMA,
        pltpu.SemaphoreType.DMA,
    ],
)(_sc_body)


def _tc_body(p_ref, w_ref, o_ref):
    x = p_ref[0] + p_ref[1]
    o_ref[...] = jnp.maximum(
        jnp.dot(x, w_ref[...], preferred_element_type=jnp.float32), 0.0)


M_BLK = 1000

_tc_matmul = pl.pallas_call(
    _tc_body,
    grid=(N_NODES // M_BLK,),
    in_specs=[pl.BlockSpec((NC, M_BLK, D), lambda m: (0, m, 0)),
              pl.BlockSpec((D, D), lambda m: (0, 0))],
    out_specs=pl.BlockSpec((M_BLK, D), lambda m: (m, 0)),
    out_shape=jax.ShapeDtypeStruct((N_NODES, D), jnp.float32),
)


def kernel(feat, edge_index, edge_weight, W):
    src = edge_index[0].astype(jnp.int32)
    dst = edge_index[1].astype(jnp.int32)
    pad = E_PAD - N_EDGES
    spread = (jnp.arange(pad, dtype=jnp.int32) * 8) % N_NODES
    src = jnp.concatenate([src, spread]).reshape(NW, NBLK, NBCH, K)
    dst = jnp.concatenate([dst, spread]).reshape(NW, NBLK, NBCH, K)
    w4 = jnp.concatenate(
        [edge_weight, jnp.zeros((pad,), jnp.float32)]).reshape(NW, NBLK, NBCH, K)
    partials = _sc_scatter(feat, src, dst, w4)
    return _tc_matmul(partials.reshape(NC, N_NODES, D), W)
